# baseline (speedup 1.0000x reference)
import functools

import jax
import jax.numpy as jnp
from jax import lax
from jax.experimental import pallas as pl
from jax.experimental.pallas import tpu as pltpu
from jax.experimental.pallas import tpu_sc as plsc


@functools.cache
def _build(B, D, VU, VF):
    info = plsc.get_sparse_core_info()
    NW = info.num_cores * info.num_subcores
    NC = info.num_cores
    b_per_w = B // NW

    mesh = plsc.VectorSubcoreMesh(core_axis_name="c", subcore_axis_name="s")

    @functools.partial(
        pl.kernel,
        mesh=mesh,
        out_type=jax.ShapeDtypeStruct((2 * D, B), jnp.float32),
        compiler_params=pltpu.CompilerParams(use_tc_tiling_on_sc=False),
        scratch_types=[
            pltpu.VMEM((b_per_w,), jnp.int32),
            pltpu.VMEM((b_per_w,), jnp.int32),
            pltpu.VMEM((2 * D, b_per_w), jnp.float32),
            pltpu.SemaphoreType.DMA,
        ],
    )
    def k(uid_hbm, eid_hbm, utT_hbm, ftT_hbm, outT_hbm,
          uidx_v, fidx_v, blk_v, sem):
        wid = lax.axis_index("s") * NC + lax.axis_index("c")
        base = wid * b_per_w
        pltpu.sync_copy(uid_hbm.at[pl.ds(base, b_per_w)], uidx_v)
        pltpu.sync_copy(eid_hbm.at[pl.ds(base, b_per_w)], fidx_v)
        copies = []
        for j in range(D):
            copies.append(pltpu.async_copy(
                utT_hbm.at[j].at[uidx_v], blk_v.at[j], sem))
            copies.append(pltpu.async_copy(
                ftT_hbm.at[j].at[fidx_v], blk_v.at[D + j], sem))
        for c in copies:
            c.wait()
        pltpu.sync_copy(blk_v, outT_hbm.at[:, pl.ds(base, b_per_w)])

    return k


def kernel(userId, emotionId, user_table, feeling_table):
    B = userId.shape[0]
    VU, D = user_table.shape
    VF = feeling_table.shape[0]
    VUp = (VU + 7) // 8 * 8
    VFp = (VF + 7) // 8 * 8
    utT = jnp.pad(user_table.T, ((0, 0), (0, VUp - VU)))
    ftT = jnp.pad(feeling_table.T, ((0, 0), (0, VFp - VF)))
    outT = _build(B, D, VUp, VFp)(userId, emotionId, utT, ftT)
    return outT.T


# speedup vs baseline: 4.8808x; 4.8808x over previous
import functools

import jax
import jax.numpy as jnp
from jax import lax
from jax.experimental import pallas as pl
from jax.experimental.pallas import tpu as pltpu
from jax.experimental.pallas import tpu_sc as plsc


@functools.cache
def _build(B, D, VUp, VFp):
    info = plsc.get_sparse_core_info()
    NW = info.num_cores * info.num_subcores
    NC = info.num_cores
    b_per_w = B // NW
    CH = 256
    n_ch = b_per_w // CH

    mesh = plsc.VectorSubcoreMesh(core_axis_name="c", subcore_axis_name="s")

    @functools.partial(
        pl.kernel,
        mesh=mesh,
        out_type=jax.ShapeDtypeStruct((B, 128), jnp.float32),
        scratch_types=[
            pltpu.VMEM((b_per_w,), jnp.int32),
            pltpu.VMEM((b_per_w,), jnp.int32),
            pltpu.VMEM((CH, 128), jnp.float32),
            pltpu.VMEM((CH, 128), jnp.float32),
            pltpu.SemaphoreType.DMA,
            pltpu.SemaphoreType.DMA,
        ],
    )
    def k(uid_hbm, eid_hbm, up_hbm, fp_hbm, out_hbm,
          uidx_v, fidx_v, ublk_v, fblk_v, sem_u, sem_f):
        wid = lax.axis_index("s") * NC + lax.axis_index("c")
        base = wid * b_per_w
        pltpu.sync_copy(uid_hbm.at[pl.ds(base, b_per_w)], uidx_v)
        pltpu.sync_copy(eid_hbm.at[pl.ds(base, b_per_w)], fidx_v)
        for c in range(n_ch):
            cu = pltpu.async_copy(
                up_hbm.at[uidx_v.at[pl.ds(c * CH, CH)]], ublk_v, sem_u)
            cf = pltpu.async_copy(
                fp_hbm.at[fidx_v.at[pl.ds(c * CH, CH)]], fblk_v, sem_f)
            cu.wait()
            cf.wait()

            def merge_row(i, _):
                ublk_v[i, pl.ds(D, 16)] = fblk_v[i, pl.ds(0, 16)]
                ublk_v[i, pl.ds(D + 16, 16)] = fblk_v[i, pl.ds(16, 16)]
                return 0

            lax.fori_loop(0, CH, merge_row, 0, unroll=8)
            pltpu.sync_copy(ublk_v,
                            out_hbm.at[pl.ds(base + c * CH, CH)])

    return k


def kernel(userId, emotionId, user_table, feeling_table):
    B = userId.shape[0]
    VU, D = user_table.shape
    VF = feeling_table.shape[0]
    VUp = (VU + 7) // 8 * 8
    VFp = (VF + 7) // 8 * 8
    up = jnp.pad(user_table, ((0, VUp - VU), (0, 128 - D)))
    fp = jnp.pad(feeling_table, ((0, VFp - VF), (0, 128 - D)))
    out = _build(B, D, VUp, VFp)(userId, emotionId, up, fp)
    return out[:, : 2 * D]


# speedup vs baseline: 6.5045x; 1.3327x over previous
import functools

import jax
import jax.numpy as jnp
from jax import lax
from jax.experimental import pallas as pl
from jax.experimental.pallas import tpu as pltpu
from jax.experimental.pallas import tpu_sc as plsc


@functools.cache
def _build(B, D, VU, VF):
    info = plsc.get_sparse_core_info()
    NW = info.num_cores * info.num_subcores
    NC = info.num_cores
    b_per_w = B // NW
    K = 16

    mesh = plsc.VectorSubcoreMesh(core_axis_name="c", subcore_axis_name="s")

    @functools.partial(
        pl.kernel,
        mesh=mesh,
        out_type=jax.ShapeDtypeStruct((B, 128), jnp.float32),
        scratch_types=[
            pltpu.VMEM((b_per_w,), jnp.int32),
            pltpu.VMEM((b_per_w,), jnp.int32),
            pltpu.VMEM((K, 8, D), jnp.float32),
            pltpu.VMEM((K, 8, D), jnp.float32),
            pltpu.VMEM((b_per_w, 128), jnp.float32),
            pltpu.SemaphoreType.DMA,
        ],
    )
    def k(uid_hbm, eid_hbm, ut_hbm, ft_hbm, out_hbm,
          uidx_v, fidx_v, ubuf_v, fbuf_v, blk_v, sem):
        wid = lax.axis_index("s") * NC + lax.axis_index("c")
        base = wid * b_per_w
        pltpu.sync_copy(uid_hbm.at[pl.ds(base, b_per_w)], uidx_v)
        pltpu.sync_copy(eid_hbm.at[pl.ds(base, b_per_w)], fidx_v)

        def batch(c, _):
            uvec = uidx_v[pl.ds(c * K, K)]
            fvec = fidx_v[pl.ds(c * K, K)]
            for n in range(K):
                pltpu.async_copy(
                    ut_hbm.at[
                        pl.ds(pl.multiple_of((uvec[n] >> 3) * 8, 8), 8)],
                    ubuf_v.at[n], sem)
                pltpu.async_copy(
                    ft_hbm.at[
                        pl.ds(pl.multiple_of((fvec[n] >> 3) * 8, 8), 8)],
                    fbuf_v.at[n], sem)

            def drain(n, _):
                pltpu.make_async_copy(
                    ut_hbm.at[pl.ds(0, 8)], ubuf_v.at[0], sem).wait()
                pltpu.make_async_copy(
                    ft_hbm.at[pl.ds(0, 8)], fbuf_v.at[0], sem).wait()
                return 0

            lax.fori_loop(0, K, drain, 0)

            ruv = uvec & 7
            rfv = fvec & 7
            for n in range(K):
                i = c * K + n
                ru = ruv[n]
                rf = rfv[n]
                blk_v[i, pl.ds(0, 16)] = ubuf_v[n, ru, pl.ds(0, 16)]
                blk_v[i, pl.ds(16, 16)] = ubuf_v[n, ru, pl.ds(16, 16)]
                blk_v[i, pl.ds(D, 16)] = fbuf_v[n, rf, pl.ds(0, 16)]
                blk_v[i, pl.ds(D + 16, 16)] = fbuf_v[n, rf, pl.ds(16, 16)]
            return 0

        lax.fori_loop(0, b_per_w // K, batch, 0)
        pltpu.sync_copy(blk_v, out_hbm.at[pl.ds(base, b_per_w)])

    return k


def kernel(userId, emotionId, user_table, feeling_table):
    B = userId.shape[0]
    VU, D = user_table.shape
    VF = feeling_table.shape[0]
    out = _build(B, D, VU, VF)(userId, emotionId, user_table, feeling_table)
    return out[:, : 2 * D]


# speedup vs baseline: 7.2040x; 1.1076x over previous
import functools

import jax
import jax.numpy as jnp
from jax import lax
from jax.experimental import pallas as pl
from jax.experimental.pallas import tpu as pltpu
from jax.experimental.pallas import tpu_sc as plsc


@functools.cache
def _build(B, D, VU, VF):
    info = plsc.get_sparse_core_info()
    NW = info.num_cores * info.num_subcores
    NC = info.num_cores
    b_per_w = B // NW
    K = 16

    mesh = plsc.VectorSubcoreMesh(core_axis_name="c", subcore_axis_name="s")

    @functools.partial(
        pl.kernel,
        mesh=mesh,
        out_type=jax.ShapeDtypeStruct((B, 128), jnp.float32),
        scratch_types=[
            pltpu.VMEM((b_per_w,), jnp.int32),
            pltpu.VMEM((b_per_w,), jnp.int32),
            pltpu.VMEM((K, D), jnp.float32),
            pltpu.VMEM((K, D), jnp.float32),
            pltpu.VMEM((b_per_w, 128), jnp.float32),
            pltpu.SemaphoreType.DMA,
        ],
    )
    def k(uid_hbm, eid_hbm, ut_hbm, ft_hbm, out_hbm,
          uidx_v, fidx_v, ubuf_v, fbuf_v, blk_v, sem):
        wid = lax.axis_index("s") * NC + lax.axis_index("c")
        base = wid * b_per_w
        pltpu.sync_copy(uid_hbm.at[pl.ds(base, b_per_w)], uidx_v)
        pltpu.sync_copy(eid_hbm.at[pl.ds(base, b_per_w)], fidx_v)

        def batch(c, _):
            uvec = uidx_v[pl.ds(c * K, K)]
            fvec = fidx_v[pl.ds(c * K, K)]
            for n in range(K):
                pltpu.async_copy(ut_hbm.at[uvec[n]], ubuf_v.at[n], sem)
                pltpu.async_copy(ft_hbm.at[fvec[n]], fbuf_v.at[n], sem)

            def drain(n, _):
                pltpu.make_async_copy(
                    ut_hbm.at[0], ubuf_v.at[0], sem).wait()
                pltpu.make_async_copy(
                    ft_hbm.at[0], fbuf_v.at[0], sem).wait()
                return 0

            lax.fori_loop(0, K, drain, 0)

            for n in range(K):
                i = c * K + n
                blk_v[i, pl.ds(0, 16)] = ubuf_v[n, pl.ds(0, 16)]
                blk_v[i, pl.ds(16, 16)] = ubuf_v[n, pl.ds(16, 16)]
                blk_v[i, pl.ds(D, 16)] = fbuf_v[n, pl.ds(0, 16)]
                blk_v[i, pl.ds(D + 16, 16)] = fbuf_v[n, pl.ds(16, 16)]
            return 0

        lax.fori_loop(0, b_per_w // K, batch, 0)
        pltpu.sync_copy(blk_v, out_hbm.at[pl.ds(base, b_per_w)])

    return k


def kernel(userId, emotionId, user_table, feeling_table):
    B = userId.shape[0]
    VU, D = user_table.shape
    VF = feeling_table.shape[0]
    out = _build(B, D, VU, VF)(userId, emotionId, user_table, feeling_table)
    return out[:, : 2 * D]


# speedup vs baseline: 7.5183x; 1.0436x over previous
import functools

import jax
import jax.numpy as jnp
from jax import lax
from jax.experimental import pallas as pl
from jax.experimental.pallas import tpu as pltpu
from jax.experimental.pallas import tpu_sc as plsc


@functools.cache
def _build(B, D, VU, VF):
    info = plsc.get_sparse_core_info()
    NW = info.num_cores * info.num_subcores
    NC = info.num_cores
    b_per_w = B // NW
    K = 64

    mesh = plsc.VectorSubcoreMesh(core_axis_name="c", subcore_axis_name="s")

    @functools.partial(
        pl.kernel,
        mesh=mesh,
        out_type=jax.ShapeDtypeStruct((B, 128), jnp.float32),
        scratch_types=[
            pltpu.VMEM((b_per_w,), jnp.int32),
            pltpu.VMEM((b_per_w,), jnp.int32),
            pltpu.VMEM((K, D), jnp.float32),
            pltpu.VMEM((K, D), jnp.float32),
            pltpu.VMEM((b_per_w, 128), jnp.float32),
            pltpu.SemaphoreType.DMA,
        ],
    )
    def k(uid_hbm, eid_hbm, ut_hbm, ft_hbm, out_hbm,
          uidx_v, fidx_v, ubuf_v, fbuf_v, blk_v, sem):
        wid = lax.axis_index("s") * NC + lax.axis_index("c")
        base = wid * b_per_w
        pltpu.sync_copy(uid_hbm.at[pl.ds(base, b_per_w)], uidx_v)
        pltpu.sync_copy(eid_hbm.at[pl.ds(base, b_per_w)], fidx_v)

        def batch(c, _):
            uvec = uidx_v[pl.ds(c * K, K)]
            fvec = fidx_v[pl.ds(c * K, K)]
            for n in range(K):
                pltpu.async_copy(ut_hbm.at[uvec[n]], ubuf_v.at[n], sem)
                pltpu.async_copy(ft_hbm.at[fvec[n]], fbuf_v.at[n], sem)

            def drain(n, _):
                pltpu.make_async_copy(
                    ut_hbm.at[0], ubuf_v.at[0], sem).wait()
                pltpu.make_async_copy(
                    ft_hbm.at[0], fbuf_v.at[0], sem).wait()
                return 0

            lax.fori_loop(0, K, drain, 0)

            for n in range(K):
                i = c * K + n
                blk_v[i, pl.ds(0, 16)] = ubuf_v[n, pl.ds(0, 16)]
                blk_v[i, pl.ds(16, 16)] = ubuf_v[n, pl.ds(16, 16)]
                blk_v[i, pl.ds(D, 16)] = fbuf_v[n, pl.ds(0, 16)]
                blk_v[i, pl.ds(D + 16, 16)] = fbuf_v[n, pl.ds(16, 16)]
            return 0

        lax.fori_loop(0, b_per_w // K, batch, 0)
        pltpu.sync_copy(blk_v, out_hbm.at[pl.ds(base, b_per_w)])

    return k


def kernel(userId, emotionId, user_table, feeling_table):
    B = userId.shape[0]
    VU, D = user_table.shape
    VF = feeling_table.shape[0]
    out = _build(B, D, VU, VF)(userId, emotionId, user_table, feeling_table)
    return out[:, : 2 * D]
